# Initial kernel scaffold; baseline (speedup 1.0000x reference)
import jax
import jax.numpy as jnp
from jax.experimental import pallas as pl


def kernel(x, edge_index, W1_00, b1_00, W2_00, b2_00, W1_10, b1_10, W2_10, b2_10, W1_11, b1_11, W2_11, b2_11):
    raise NotImplementedError("write your pallas kernel here")


# speedup vs baseline: 2.2496x; 2.2496x over previous
import functools

import jax
import jax.numpy as jnp
from jax import lax
from jax.experimental import pallas as pl
from jax.experimental.pallas import tpu as pltpu
from jax.experimental.pallas import tpu_sc as plsc

N = 10000
E = 320000
D = 128

NC, NS, L = 2, 16, 16
NW = NC * NS
NPAD = 10240
CH = 128
EPAD = 323584
PER_W = EPAD // NW
NCHUNK = PER_W // CH
ROWS_PER_S = NPAD // NS

_mesh = plsc.VectorSubcoreMesh(core_axis_name="c", subcore_axis_name="s")


def _spmm_body(sup, srcp, dstp, out, srcv, dstv, rows, acc, sem):
    c = lax.axis_index("c")
    s = lax.axis_index("s")
    wid = s * NC + c

    zeros = jnp.zeros((L,), jnp.float32)

    def zbody(r, carry):
        for k in range(D // L):
            rows[r, pl.ds(k * L, L)] = zeros
        return carry

    lax.fori_loop(0, CH, zbody, 0)
    rbase = s * ROWS_PER_S
    for k in range(ROWS_PER_S // CH):
        pltpu.sync_copy(rows, acc.at[pl.ds(rbase + k * CH, CH)])
    plsc.subcore_barrier()

    ebase = wid * PER_W

    def ebody(i, carry):
        off = ebase + i * CH
        pltpu.sync_copy(srcp.at[pl.ds(off, CH)], srcv)
        pltpu.sync_copy(dstp.at[pl.ds(off, CH)], dstv)
        pltpu.async_copy(sup.at[srcv], rows, sem).wait()
        pltpu.sync_copy(rows, acc.at[dstv], add=True)
        return carry

    lax.fori_loop(0, NCHUNK, ebody, 0)
    plsc.subcore_barrier()
    pltpu.sync_copy(acc.at[pl.ds(rbase, ROWS_PER_S)],
                    out.at[c, pl.ds(rbase, ROWS_PER_S)])


_spmm = functools.partial(
    pl.kernel,
    out_type=jax.ShapeDtypeStruct((NC, NPAD, D), jnp.float32),
    mesh=_mesh,
    scratch_types=[
        pltpu.VMEM((CH,), jnp.int32),
        pltpu.VMEM((CH,), jnp.int32),
        pltpu.VMEM((CH, D), jnp.float32),
        pltpu.VMEM_SHARED((NPAD, D), jnp.float32),
        pltpu.SemaphoreType.DMA,
    ],
)(_spmm_body)


BM = 2000


def _row_spec(i_map=lambda i: (i, 0), shape=None):
    return pl.BlockSpec(shape, i_map)


def _tc1_body(x_ref, wa_ref, wb_ref, oa_ref, ob_ref):
    xv = x_ref[...]
    oa_ref[...] = jnp.dot(xv, wa_ref[...], preferred_element_type=jnp.float32)
    ob_ref[...] = jnp.dot(xv, wb_ref[...], preferred_element_type=jnp.float32)


def _tc1(x, wa, wb):
    return pl.pallas_call(
        _tc1_body,
        grid=(N // BM,),
        in_specs=[pl.BlockSpec((BM, D), lambda i: (i, 0)),
                  pl.BlockSpec((D, D), lambda i: (0, 0)),
                  pl.BlockSpec((D, D), lambda i: (0, 0))],
        out_specs=[pl.BlockSpec((BM, D), lambda i: (i, 0))] * 2,
        out_shape=[jax.ShapeDtypeStruct((N, D), jnp.float32)] * 2,
    )(x, wa, wb)


def _tc2_body(p0a, p1a, ba, wa, p0b, p1b, bb, wb, oa, ob):
    ha = jnp.maximum(p0a[...] + p1a[...] + ba[...], 0.0)
    oa[...] = jnp.dot(ha, wa[...], preferred_element_type=jnp.float32)
    hb = jnp.maximum(p0b[...] + p1b[...] + bb[...], 0.0)
    ob[...] = jnp.dot(hb, wb[...], preferred_element_type=jnp.float32)


def _tc2(p0a, p1a, ba, wa, p0b, p1b, bb, wb):
    pspec = pl.BlockSpec((BM, D), lambda i: (i, 0))
    bspec = pl.BlockSpec((1, D), lambda i: (0, 0))
    wspec = pl.BlockSpec((D, D), lambda i: (0, 0))
    return pl.pallas_call(
        _tc2_body,
        grid=(N // BM,),
        in_specs=[pspec, pspec, bspec, wspec, pspec, pspec, bspec, wspec],
        out_specs=[pl.BlockSpec((BM, D), lambda i: (i, 0))] * 2,
        out_shape=[jax.ShapeDtypeStruct((N, D), jnp.float32)] * 2,
    )(p0a, p1a, ba, wa, p0b, p1b, bb, wb)


def _tc3_body(p0, p1, b, w, o):
    y = p0[...] + p1[...] + b[...]
    nrm = jnp.maximum(jnp.sqrt(jnp.sum(y * y, axis=1, keepdims=True)), 1e-12)
    o[...] = jnp.dot(y / nrm, w[...], preferred_element_type=jnp.float32)


def _tc3(p0, p1, b, w):
    pspec = pl.BlockSpec((BM, D), lambda i: (i, 0))
    return pl.pallas_call(
        _tc3_body,
        grid=(N // BM,),
        in_specs=[pspec, pspec, pl.BlockSpec((1, D), lambda i: (0, 0)),
                  pl.BlockSpec((D, D), lambda i: (0, 0))],
        out_specs=pl.BlockSpec((BM, D), lambda i: (i, 0)),
        out_shape=jax.ShapeDtypeStruct((N, D), jnp.float32),
    )(p0, p1, b, w)


def _tc4_body(p0, p1, b, w, o):
    h = jnp.maximum(p0[...] + p1[...] + b[...], 0.0)
    o[...] = jnp.dot(h, w[...], preferred_element_type=jnp.float32)


def _tc4(p0, p1, b, w):
    pspec = pl.BlockSpec((BM, D), lambda i: (i, 0))
    return pl.pallas_call(
        _tc4_body,
        grid=(N // BM,),
        in_specs=[pspec, pspec, pl.BlockSpec((1, D), lambda i: (0, 0)),
                  pl.BlockSpec((D, D), lambda i: (0, 0))],
        out_specs=pl.BlockSpec((BM, D), lambda i: (i, 0)),
        out_shape=jax.ShapeDtypeStruct((N, D), jnp.float32),
    )(p0, p1, b, w)


def _tc5_body(x_ref, pa0, pa1, ba, q0, q1, bq, o):
    x = x_ref[...]
    ya = pa0[...] + pa1[...] + ba[...]
    na = jnp.maximum(jnp.sqrt(jnp.sum(ya * ya, axis=1, keepdims=True)), 1e-12)
    subx0 = ya / na
    yq = q0[...] + q1[...] + bq[...]
    nq = jnp.maximum(jnp.sqrt(jnp.sum(yq * yq, axis=1, keepdims=True)), 1e-12)
    subx1 = yq / nq
    s01 = (jnp.sum(x * x, axis=1, keepdims=True)
           + jnp.sum(subx0 * subx0, axis=1, keepdims=True))
    n1 = jnp.maximum(jnp.sqrt(s01), 1e-12)
    n2 = jnp.maximum(jnp.sqrt(s01 / (n1 * n1)
                              + jnp.sum(subx1 * subx1, axis=1, keepdims=True)),
                     1e-12)
    o[...] = jnp.concatenate(
        [x / (n1 * n2), subx0 / (n1 * n2), subx1 / n2], axis=1)


def _tc5(x, pa0, pa1, ba, q0, q1, bq):
    pspec = pl.BlockSpec((BM, D), lambda i: (i, 0))
    bspec = pl.BlockSpec((1, D), lambda i: (0, 0))
    return pl.pallas_call(
        _tc5_body,
        grid=(N // BM,),
        in_specs=[pspec, pspec, pspec, bspec, pspec, pspec, bspec],
        out_specs=pl.BlockSpec((BM, 3 * D), lambda i: (i, 0)),
        out_shape=jax.ShapeDtypeStruct((N, 3 * D), jnp.float32),
    )(x, pa0, pa1, ba, q0, q1, bq)


def kernel(x, edge_index, W1_00, b1_00, W2_00, b2_00, W1_10, b1_10, W2_10,
           b2_10, W1_11, b1_11, W2_11, b2_11):
    src = edge_index[0]
    dst = edge_index[1]
    srcp = jnp.concatenate([src, jnp.zeros((EPAD - E,), jnp.int32)])
    dstp = jnp.concatenate([dst, jnp.full((EPAD - E,), N, jnp.int32)])

    ba1, bb1 = b1_00.reshape(1, D), b1_10.reshape(1, D)
    ba2, bb2 = b2_00.reshape(1, D), b2_10.reshape(1, D)

    s1a, s1b = _tc1(x, W1_00, W1_10)
    a1a = _spmm(s1a, srcp, dstp)
    a1b = _spmm(s1b, srcp, dstp)
    s2a, s2b = _tc2(a1a[0], a1a[1], ba1, W2_00, a1b[0], a1b[1], bb1, W2_10)
    a2a = _spmm(s2a, srcp, dstp)
    a2b = _spmm(s2b, srcp, dstp)
    s3 = _tc3(a2b[0], a2b[1], bb2, W1_11)
    p3 = _spmm(s3, srcp, dstp)
    s4 = _tc4(p3[0], p3[1], b1_11.reshape(1, D), W2_11)
    q = _spmm(s4, srcp, dstp)
    return _tc5(x, a2a[0], a2a[1], ba2, q[0], q[1], b2_11.reshape(1, D))
